# baseline (speedup 1.0000x reference)
import functools

import jax
import jax.numpy as jnp
from jax import lax
from jax.experimental import pallas as pl
from jax.experimental.pallas import tpu as pltpu
from jax.experimental.pallas import tpu_sc as plsc

D_MODEL = 1024
MAX_SEQ_LEN = 2048
BATCH = 16
TJ = 512
NCK = MAX_SEQ_LEN // TJ

_SC_INFO = plsc.get_sparse_core_info()
L = _SC_INFO.num_lanes


def _pos_sc_body(lenrep_hbm, out_hbm, len_v, pos_v):
    b = lax.axis_index("s")
    pltpu.sync_copy(lenrep_hbm.at[b], len_v)
    len_vec = len_v[...]
    for k in range(MAX_SEQ_LEN // L):
        col = k * L + lax.iota(jnp.int32, L)
        pos_v[pl.ds(k * L, L)] = jnp.where(col < len_vec, col + 1, 0)
    pltpu.sync_copy(pos_v, out_hbm.at[b])


@functools.partial(
    pl.kernel,
    out_type=jax.ShapeDtypeStruct((BATCH, MAX_SEQ_LEN), jnp.int32),
    mesh=plsc.VectorSubcoreMesh(
        core_axis_name="c", subcore_axis_name="s", num_cores=1
    ),
    scratch_types=[
        pltpu.VMEM((L,), jnp.int32),
        pltpu.VMEM((MAX_SEQ_LEN,), jnp.int32),
    ],
)
def _pos_sc(lenrep_hbm, out_hbm, len_v, pos_v):
    _pos_sc_body(lenrep_hbm, out_hbm, len_v, pos_v)


def _chunk_copy(src, out_ref, b, c, sem):
    return pltpu.make_async_copy(src, out_ref.at[b, pl.ds(c * TJ, TJ)], sem)


def _emb_body(len_ref, pe_ref, out_ref, zbuf, sbuf, sems):
    b = pl.program_id(0)
    len_b = len_ref[b]

    @pl.when(b == 0)
    def _init_zeros():
        zbuf[...] = jnp.zeros((TJ, D_MODEL), jnp.float32)

    @pl.when(b > 0)
    def _drain_prev():
        for q in range(NCK):
            _chunk_copy(zbuf, out_ref, 0, 0, sems.at[q]).wait()

    for c in range(NCK):
        lo, hi = c * TJ, (c + 1) * TJ

        @pl.when(len_b >= hi)
        def _full():
            _chunk_copy(pe_ref.at[pl.ds(lo, TJ)], out_ref, b, c, sems.at[c]).start()

        @pl.when(len_b <= lo)
        def _zero():
            _chunk_copy(zbuf, out_ref, b, c, sems.at[c]).start()

        @pl.when(jnp.logical_and(len_b > lo, len_b < hi))
        def _straddle():
            rows = lo + jax.lax.broadcasted_iota(jnp.int32, (TJ, 1), 0)
            sbuf[b % 2] = jnp.where(rows < len_b, pe_ref[pl.ds(lo, TJ)], 0.0)
            _chunk_copy(sbuf.at[b % 2], out_ref, b, c, sems.at[c]).start()

    @pl.when(b == BATCH - 1)
    def _drain_last():
        for q in range(NCK):
            _chunk_copy(zbuf, out_ref, 0, 0, sems.at[q]).wait()


def kernel(input_len, pe_table):
    lenrep = jnp.broadcast_to(input_len[:, None], (BATCH, L))
    pos = _pos_sc(lenrep)

    pe = pe_table[1:]
    emb = pl.pallas_call(
        _emb_body,
        grid=(BATCH,),
        in_specs=[
            pl.BlockSpec(memory_space=pltpu.SMEM),
            pl.BlockSpec((MAX_SEQ_LEN, D_MODEL), lambda b: (0, 0)),
        ],
        out_specs=pl.BlockSpec(memory_space=pl.ANY),
        out_shape=jax.ShapeDtypeStruct((BATCH, MAX_SEQ_LEN, D_MODEL), jnp.float32),
        scratch_shapes=[
            pltpu.VMEM((TJ, D_MODEL), jnp.float32),
            pltpu.VMEM((2, TJ, D_MODEL), jnp.float32),
            pltpu.SemaphoreType.DMA((NCK,)),
        ],
    )(input_len, pe)
    return (emb, pos)


# speedup vs baseline: 1.1546x; 1.1546x over previous
import functools

import jax
import jax.numpy as jnp
from jax import lax
from jax.experimental import pallas as pl
from jax.experimental.pallas import tpu as pltpu
from jax.experimental.pallas import tpu_sc as plsc

D_MODEL = 1024
MAX_SEQ_LEN = 2048
BATCH = 16
TJ = 512
NCK = MAX_SEQ_LEN // TJ

_SC_INFO = plsc.get_sparse_core_info()
L = _SC_INFO.num_lanes


def _pos_sc_body(lenrep_hbm, out_hbm, len_v, pos_v):
    b = lax.axis_index("s")
    pltpu.sync_copy(lenrep_hbm.at[b], len_v)
    len_vec = len_v[...]
    for k in range(MAX_SEQ_LEN // L):
        col = k * L + lax.iota(jnp.int32, L)
        pos_v[pl.ds(k * L, L)] = jnp.where(col < len_vec, col + 1, 0)
    pltpu.sync_copy(pos_v, out_hbm.at[b])


@functools.partial(
    pl.kernel,
    out_type=jax.ShapeDtypeStruct((BATCH, MAX_SEQ_LEN), jnp.int32),
    mesh=plsc.VectorSubcoreMesh(
        core_axis_name="c", subcore_axis_name="s", num_cores=1
    ),
    scratch_types=[
        pltpu.VMEM((L,), jnp.int32),
        pltpu.VMEM((MAX_SEQ_LEN,), jnp.int32),
    ],
)
def _pos_sc(lenrep_hbm, out_hbm, len_v, pos_v):
    _pos_sc_body(lenrep_hbm, out_hbm, len_v, pos_v)


def _chunk_copy(src, out_ref, b, c, sem):
    return pltpu.make_async_copy(src, out_ref.at[b, pl.ds(c * TJ, TJ)], sem)


def _emb_body(len_ref, pe_ref, out_ref, zbuf, sbuf, sems):
    b = pl.program_id(0)
    len_b = len_ref[b]

    @pl.when(b == 0)
    def _init_zeros():
        zbuf[...] = jnp.zeros((TJ, D_MODEL), jnp.float32)

    @pl.when(b > 1)
    def _drain_prev():
        for q in range(NCK):
            _chunk_copy(zbuf, out_ref, 0, 0, sems.at[q]).wait()

    for c in range(NCK):
        lo, hi = c * TJ, (c + 1) * TJ

        @pl.when(len_b >= hi)
        def _full():
            _chunk_copy(pe_ref.at[pl.ds(lo, TJ)], out_ref, b, c, sems.at[c]).start()

        @pl.when(len_b <= lo)
        def _zero():
            _chunk_copy(zbuf, out_ref, b, c, sems.at[c]).start()

        @pl.when(jnp.logical_and(len_b > lo, len_b < hi))
        def _straddle():
            rows = lo + jax.lax.broadcasted_iota(jnp.int32, (TJ, 1), 0)
            sbuf[b % 2] = jnp.where(rows < len_b, pe_ref[pl.ds(lo, TJ)], 0.0)
            _chunk_copy(sbuf.at[b % 2], out_ref, b, c, sems.at[c]).start()

    @pl.when(b == BATCH - 1)
    def _drain_last():
        for _ in range(2):
            for q in range(NCK):
                _chunk_copy(zbuf, out_ref, 0, 0, sems.at[q]).wait()


def kernel(input_len, pe_table):
    lenrep = jnp.broadcast_to(input_len[:, None], (BATCH, L))
    pos = _pos_sc(lenrep)

    pe = pe_table[1:]
    emb = pl.pallas_call(
        _emb_body,
        grid=(BATCH,),
        in_specs=[
            pl.BlockSpec(memory_space=pltpu.SMEM),
            pl.BlockSpec((MAX_SEQ_LEN, D_MODEL), lambda b: (0, 0)),
        ],
        out_specs=pl.BlockSpec(memory_space=pl.ANY),
        out_shape=jax.ShapeDtypeStruct((BATCH, MAX_SEQ_LEN, D_MODEL), jnp.float32),
        scratch_shapes=[
            pltpu.VMEM((TJ, D_MODEL), jnp.float32),
            pltpu.VMEM((2, TJ, D_MODEL), jnp.float32),
            pltpu.SemaphoreType.DMA((NCK,)),
        ],
    )(input_len, pe)
    return (emb, pos)
